# baseline (speedup 1.0000x reference)
import jax
import jax.numpy as jnp
from jax import lax
from jax.experimental import pallas as pl
from jax.experimental.pallas import tpu as pltpu

_B = 128
_K = 32768
_C = 128
_S = 64
_BK = 8192
_NK = _K // _BK


def _tc_body(x_ref, w_ref, b_ref, logits_ref, out_ref, acc_ref, x128_ref):
    k = pl.program_id(0)

    @pl.when(k == 0)
    def _():
        acc_ref[...] = jnp.zeros_like(acc_ref)
        x128_ref[...] = x_ref[:, :_C]

    acc_ref[...] += lax.dot_general(
        x_ref[...], w_ref[...],
        dimension_numbers=(((1,), (1,)), ((), ())),
        preferred_element_type=jnp.float32,
    )

    @pl.when(k == _NK - 1)
    def _():
        h = acc_ref[...] + b_ref[...]
        m = jnp.max(h, axis=1, keepdims=True)
        e = jnp.exp(h - m)
        p = e / jnp.sum(e, axis=1, keepdims=True)
        logits_ref[...] = p
        x128 = x128_ref[...]

        iu = lax.broadcasted_iota(jnp.int32, (_B // 2, _B), 0)
        ij = lax.broadcasted_iota(jnp.int32, (_B // 2, _B), 1)
        lanes = lax.broadcasted_iota(jnp.int32, (_B // 2, _C), 1)

        def pick(mat, arr):
            return lax.dot_general(
                mat, arr,
                dimension_numbers=(((1,), (0,)), ((), ())),
                precision=lax.Precision.HIGHEST,
                preferred_element_type=jnp.float32,
            )

        halves = []
        for off in (0, 1):
            sel_mat = (ij == 2 * iu + off).astype(jnp.float32)
            pk = pick(sel_mat, p)
            vk = pick(sel_mat, x128)
            ik = lanes
            for kk in (2, 4, 8, 16, 32, 64, 128):
                jj = kk // 2
                while jj >= 1:
                    pl_ = jnp.concatenate([pk[:, jj:], pk[:, :jj]], axis=1)
                    pr_ = jnp.concatenate([pk[:, -jj:], pk[:, :-jj]], axis=1)
                    il_ = jnp.concatenate([ik[:, jj:], ik[:, :jj]], axis=1)
                    ir_ = jnp.concatenate([ik[:, -jj:], ik[:, :-jj]], axis=1)
                    vl_ = jnp.concatenate([vk[:, jj:], vk[:, :jj]], axis=1)
                    vr_ = jnp.concatenate([vk[:, -jj:], vk[:, :-jj]], axis=1)
                    low = (lanes & jj) == 0
                    pp = jnp.where(low, pl_, pr_)
                    ip = jnp.where(low, il_, ir_)
                    vp = jnp.where(low, vl_, vr_)
                    m_ = jnp.logical_or(
                        pk > pp,
                        jnp.logical_and(pk == pp, ik < ip))
                    flip = jnp.logical_xor((lanes & kk) == 0, low)
                    keep = jnp.logical_xor(m_, flip)
                    pk = jnp.where(keep, pk, pp)
                    ik = jnp.where(keep, ik, ip)
                    vk = jnp.where(keep, vk, vp)
                    jj //= 2
            halves.append(vk[:, :_S])

        tu = lax.broadcasted_iota(jnp.int32, (_B, _B // 2), 0)
        tj = lax.broadcasted_iota(jnp.int32, (_B, _B // 2), 1)
        out = jnp.zeros((_B, _S), jnp.float32)
        for off, g in zip((0, 1), halves):
            back = (tu == 2 * tj + off).astype(jnp.float32)
            out = out + lax.dot_general(
                back, g,
                dimension_numbers=(((1,), (0,)), ((), ())),
                precision=lax.Precision.HIGHEST,
                preferred_element_type=jnp.float32,
            )
        out_ref[...] = out


_tc_call = pl.pallas_call(
    _tc_body,
    grid=(_NK,),
    in_specs=[
        pl.BlockSpec((_B, _BK), lambda k: (0, k)),
        pl.BlockSpec((_C, _BK), lambda k: (0, k)),
        pl.BlockSpec((1, _C), lambda k: (0, 0)),
    ],
    out_specs=[
        pl.BlockSpec((_B, _C), lambda k: (0, 0)),
        pl.BlockSpec((_B, _S), lambda k: (0, 0)),
    ],
    out_shape=[
        jax.ShapeDtypeStruct((_B, _C), jnp.float32),
        jax.ShapeDtypeStruct((_B, _S), jnp.float32),
    ],
    scratch_shapes=[
        pltpu.VMEM((_B, _C), jnp.float32),
        pltpu.VMEM((_B, _C), jnp.float32),
    ],
    compiler_params=pltpu.CompilerParams(
        dimension_semantics=("arbitrary",),
    ),
)


def kernel(x, W, b):
    logits, gathered = _tc_call(x, W, b.reshape(1, _C))
    return logits, gathered


# speedup vs baseline: 1.1038x; 1.1038x over previous
import jax
import jax.numpy as jnp
from jax import lax
from jax.experimental import pallas as pl
from jax.experimental.pallas import tpu as pltpu

_B = 128
_K = 32768
_C = 128
_S = 64
_BK = 8192
_NK = _K // _BK


def _tc_body(x_ref, w_ref, b_ref, logits_ref, out_ref, acc_ref, x128_ref):
    k = pl.program_id(0)

    @pl.when(k == 0)
    def _():
        acc_ref[...] = jnp.zeros_like(acc_ref)
        x128_ref[...] = x_ref[:, :_C]

    acc_ref[...] += lax.dot_general(
        x_ref[...], w_ref[...],
        dimension_numbers=(((1,), (1,)), ((), ())),
        preferred_element_type=jnp.float32,
    )

    @pl.when(k == _NK - 1)
    def _():
        h = acc_ref[...] + b_ref[...]
        m = jnp.max(h, axis=1, keepdims=True)
        e = jnp.exp(h - m)
        p = e / jnp.sum(e, axis=1, keepdims=True)
        logits_ref[...] = p
        x128 = x128_ref[...]

        iu = lax.broadcasted_iota(jnp.int32, (_B // 2, _B), 0)
        ij = lax.broadcasted_iota(jnp.int32, (_B // 2, _B), 1)
        lanes = lax.broadcasted_iota(jnp.int32, (_B // 2, _C), 1)

        def pick(mat, arr):
            return lax.dot_general(
                mat, arr,
                dimension_numbers=(((1,), (0,)), ((), ())),
                precision=lax.Precision.HIGHEST,
                preferred_element_type=jnp.float32,
            )

        halves = []
        for off in (0, 1):
            sel_mat = (ij == 2 * iu + off).astype(jnp.float32)
            pk = pick(sel_mat, p)
            vk = pick(sel_mat, x128)
            ik = lanes
            for kk in (2, 4, 8, 16, 32, 64, 128):
                jj = kk // 2
                while jj >= 1:
                    pl_ = jnp.concatenate([pk[:, jj:], pk[:, :jj]], axis=1)
                    pr_ = jnp.concatenate([pk[:, -jj:], pk[:, :-jj]], axis=1)
                    il_ = jnp.concatenate([ik[:, jj:], ik[:, :jj]], axis=1)
                    ir_ = jnp.concatenate([ik[:, -jj:], ik[:, :-jj]], axis=1)
                    vl_ = jnp.concatenate([vk[:, jj:], vk[:, :jj]], axis=1)
                    vr_ = jnp.concatenate([vk[:, -jj:], vk[:, :-jj]], axis=1)
                    low = (lanes & jj) == 0
                    pp = jnp.where(low, pl_, pr_)
                    ip = jnp.where(low, il_, ir_)
                    vp = jnp.where(low, vl_, vr_)
                    m_ = jnp.logical_or(
                        pk > pp,
                        jnp.logical_and(pk == pp, ik < ip))
                    flip = jnp.logical_xor((lanes & kk) == 0, low)
                    keep = jnp.logical_xor(m_, flip)
                    pk = jnp.where(keep, pk, pp)
                    ik = jnp.where(keep, ik, ip)
                    vk = jnp.where(keep, vk, vp)
                    jj //= 2
            halves.append(vk[:, :_S])

        tu = lax.broadcasted_iota(jnp.int32, (_B // 2, _B), 0)
        tb = lax.broadcasted_iota(jnp.int32, (_B // 2, _B), 1)
        out = jnp.zeros((_S, _B), jnp.float32)
        for off, g in zip((0, 1), halves):
            route = (tb == 2 * tu + off).astype(jnp.float32)
            out = out + lax.dot_general(
                g, route,
                dimension_numbers=(((0,), (0,)), ((), ())),
                precision=lax.Precision.HIGHEST,
                preferred_element_type=jnp.float32,
            )
        out_ref[...] = out


_tc_call = pl.pallas_call(
    _tc_body,
    grid=(_NK,),
    in_specs=[
        pl.BlockSpec((_B, _BK), lambda k: (0, k)),
        pl.BlockSpec((_C, _BK), lambda k: (0, k)),
        pl.BlockSpec((1, _C), lambda k: (0, 0)),
    ],
    out_specs=[
        pl.BlockSpec((_B, _C), lambda k: (0, 0)),
        pl.BlockSpec((_S, _B), lambda k: (0, 0)),
    ],
    out_shape=[
        jax.ShapeDtypeStruct((_B, _C), jnp.float32),
        jax.ShapeDtypeStruct((_S, _B), jnp.float32),
    ],
    scratch_shapes=[
        pltpu.VMEM((_B, _C), jnp.float32),
        pltpu.VMEM((_B, _C), jnp.float32),
    ],
    compiler_params=pltpu.CompilerParams(
        dimension_semantics=("arbitrary",),
    ),
)


def kernel(x, W, b):
    logits, gathered_t = _tc_call(x, W, b.reshape(1, _C))
    return logits, gathered_t.T
